# baseline (speedup 1.0000x reference)
import functools

import jax
import jax.numpy as jnp
from jax.experimental import pallas as pl
from jax.experimental.pallas import tpu as pltpu

B = 4096
K = 8
N = 30
IN_C = 128
F = 64
C = 5
BB = 128
GR = 512
CH = GR // K
NCH = -(-N * N * K // GR)
NB = B // BB


def _kernel(g_ref, real_ref, w1_ref, b1_ref, w2_ref, b2_ref,
            wlin_ref, blin_ref, wheadt_ref, bhead_ref,
            out1_ref, out2_ref, adj_ref):
    step = pl.program_id(0)

    @pl.when(step < NCH)
    def _phase1():
        g = g_ref[...].reshape(CH, K, B)
        s = jnp.sum(g, axis=1)
        adj_ref[step, :, :] = s.T

    @pl.when(step >= NCH)
    def _phase2():
        i = step - NCH
        bs = pl.ds(i * BB, BB)
        adj = jnp.concatenate(
            [adj_ref[c, bs, :] for c in range(NCH)], axis=1)
        adj = adj[:, : N * N].reshape(BB, N, N)

        r = real_ref[...]
        h = jax.lax.dot_general(
            r, w1_ref[...],
            dimension_numbers=(((2,), (0,)), ((), ())),
            preferred_element_type=jnp.float32)
        h = jnp.transpose(h, (1, 0, 2))

        x = jax.lax.dot_general(
            adj, h,
            dimension_numbers=(((2,), (1,)), ((0,), (0,))),
            preferred_element_type=jnp.float32)
        x = jnp.maximum(x + b1_ref[...].reshape(1, 1, F), 0.0)

        h2 = jax.lax.dot_general(
            x, w2_ref[...],
            dimension_numbers=(((2,), (0,)), ((), ())),
            preferred_element_type=jnp.float32)
        x2 = jax.lax.dot_general(
            adj, h2,
            dimension_numbers=(((2,), (1,)), ((0,), (0,))),
            preferred_element_type=jnp.float32)
        x2 = jnp.maximum(x2 + b2_ref[...].reshape(1, 1, F), 0.0)

        wlin = wlin_ref[...].reshape(1, 1, F)
        blin = blin_ref[0, 0]
        wheadt = wheadt_ref[...]
        bhead = bhead_ref[...]

        def head(xk, out_ref):
            xl = jnp.sum(xk * wlin, axis=2) + blin
            xr = jnp.maximum(xl, 0.0)
            out = jax.lax.dot_general(
                xr, wheadt,
                dimension_numbers=(((1,), (0,)), ((), ())),
                preferred_element_type=jnp.float32)
            out_ref[...] = out + bhead

        head(x, out1_ref)
        head(x2, out2_ref)


@functools.partial(jax.jit, static_argnames=())
def _run(real, graph, W1, b1, W2, b2, Wlin, blin, Whead, bhead):
    gflat = jnp.transpose(graph, (2, 3, 1, 0)).reshape(N * N * K, B)
    rT = jnp.transpose(real, (1, 0, 2))

    scale = jnp.float32(1.0 / K)
    out1, out2 = pl.pallas_call(
        _kernel,
        grid=(NCH + NB,),
        in_specs=[
            pl.BlockSpec((GR, B), lambda s: (jnp.minimum(s, NCH - 1), 0)),
            pl.BlockSpec((N, BB, IN_C),
                         lambda s: (0, jnp.maximum(s - NCH, 0), 0)),
            pl.BlockSpec((IN_C, F), lambda s: (0, 0)),
            pl.BlockSpec((1, F), lambda s: (0, 0)),
            pl.BlockSpec((F, F), lambda s: (0, 0)),
            pl.BlockSpec((1, F), lambda s: (0, 0)),
            pl.BlockSpec((1, F), lambda s: (0, 0)),
            pl.BlockSpec((1, 1), lambda s: (0, 0)),
            pl.BlockSpec((N, C), lambda s: (0, 0)),
            pl.BlockSpec((1, C), lambda s: (0, 0)),
        ],
        out_specs=[
            pl.BlockSpec((BB, C), lambda s: (jnp.maximum(s - NCH, 0), 0)),
            pl.BlockSpec((BB, C), lambda s: (jnp.maximum(s - NCH, 0), 0)),
        ],
        out_shape=[
            jax.ShapeDtypeStruct((B, C), jnp.float32),
            jax.ShapeDtypeStruct((B, C), jnp.float32),
        ],
        scratch_shapes=[pltpu.VMEM((NCH, B, CH), jnp.float32)],
        compiler_params=pltpu.CompilerParams(
            vmem_limit_bytes=60 * 1024 * 1024),
    )(gflat, rT, W1 * scale, b1.reshape(1, F), W2 * scale,
      b2.reshape(1, F), Wlin.reshape(1, F), blin.reshape(1, 1), Whead.T,
      bhead.reshape(1, C))
    return out1, out2


def kernel(real, imag, graph, W1, b1, W2, b2, Wlin, blin, Whead, bhead, layer):
    del imag
    out1, out2 = _run(real, graph, W1, b1, W2, b2, Wlin, blin, Whead, bhead)
    return jnp.where(layer > 1, out2, out1)
